# Initial kernel scaffold; baseline (speedup 1.0000x reference)
import jax
import jax.numpy as jnp
from jax import lax
from jax.experimental import pallas as pl
from jax.experimental.pallas import tpu as pltpu
from jax.experimental.pallas import tpu_sc as plsc

VOCAB = 1000000
EMBED_DIM = 32
BATCH = 16384
HIST = 50

NUM_CORES = 2
NUM_SUBCORES = 16
NUM_WORKERS = NUM_CORES * NUM_SUBCORES

TOTAL = BATCH * HIST
PER_WORKER = TOTAL // NUM_WORKERS
CHUNK = 2048

assert TOTAL % NUM_WORKERS == 0
assert PER_WORKER % CHUNK == 0, (PER_WORKER, CHUNK)


def _body(idx_hbm, table_hbm, out_hbm, idx_v, rows_v, sem):
    wid = lax.axis_index("s") * NUM_CORES + lax.axis_index("c")
    base = wid * PER_WORKER

    def chunk(g, carry):
        off = base + g * CHUNK
        pltpu.sync_copy(idx_hbm.at[pl.ds(off, CHUNK)], idx_v)
        pltpu.async_copy(table_hbm.at[idx_v], rows_v, sem).wait()
        pltpu.sync_copy(rows_v, out_hbm.at[pl.ds(off, CHUNK)])
        return carry

    lax.fori_loop(0, PER_WORKER // CHUNK, chunk, 0)


@jax.jit
def _embed(idx_flat, weight):
    mesh = plsc.VectorSubcoreMesh(
        core_axis_name="c",
        subcore_axis_name="s",
        num_cores=NUM_CORES,
        num_subcores=NUM_SUBCORES,
    )
    fn = pl.kernel(
        _body,
        out_type=jax.ShapeDtypeStruct((TOTAL, EMBED_DIM), jnp.float32),
        mesh=mesh,
        scratch_types=[
            pltpu.VMEM((CHUNK,), jnp.int32),
            pltpu.VMEM((CHUNK, EMBED_DIM), jnp.float32),
            pltpu.SemaphoreType.DMA,
        ],
    )
    return fn(idx_flat, weight)


def kernel(input_ids, weight):
    idx_flat = input_ids.reshape(-1).astype(jnp.int32)
    out = _embed(idx_flat, weight)
    return out.reshape(BATCH, HIST, EMBED_DIM)


# speedup vs baseline: 1.1099x; 1.1099x over previous
import jax
import jax.numpy as jnp
from jax import lax
from jax.experimental import pallas as pl
from jax.experimental.pallas import tpu as pltpu
from jax.experimental.pallas import tpu_sc as plsc

VOCAB = 1000000
EMBED_DIM = 32
BATCH = 16384
HIST = 50

NUM_CORES = 2
NUM_SUBCORES = 16
NUM_WORKERS = NUM_CORES * NUM_SUBCORES

TOTAL = BATCH * HIST
PER_WORKER = TOTAL // NUM_WORKERS
CHUNK = 2560

assert TOTAL % NUM_WORKERS == 0
assert PER_WORKER % CHUNK == 0, (PER_WORKER, CHUNK)


def _body(idx_hbm, table_hbm, out_hbm, idx_v, rows_v, sem):
    wid = lax.axis_index("s") * NUM_CORES + lax.axis_index("c")
    base = wid * PER_WORKER

    def chunk(g, carry):
        off = base + g * CHUNK
        pltpu.sync_copy(idx_hbm.at[pl.ds(off, CHUNK)], idx_v)
        pltpu.async_copy(table_hbm.at[idx_v], rows_v, sem).wait()
        pltpu.sync_copy(rows_v, out_hbm.at[pl.ds(off, CHUNK)])
        return carry

    lax.fori_loop(0, PER_WORKER // CHUNK, chunk, 0)


@jax.jit
def _embed(idx_flat, weight):
    mesh = plsc.VectorSubcoreMesh(
        core_axis_name="c",
        subcore_axis_name="s",
        num_cores=NUM_CORES,
        num_subcores=NUM_SUBCORES,
    )
    fn = pl.kernel(
        _body,
        out_type=jax.ShapeDtypeStruct((TOTAL, EMBED_DIM), jnp.float32),
        mesh=mesh,
        scratch_types=[
            pltpu.VMEM((CHUNK,), jnp.int32),
            pltpu.VMEM((CHUNK, EMBED_DIM), jnp.float32),
            pltpu.SemaphoreType.DMA,
        ],
        compiler_params=pltpu.CompilerParams(use_tc_tiling_on_sc=False),
    )
    return fn(idx_flat, weight)


def kernel(input_ids, weight):
    idx_flat = input_ids.reshape(-1).astype(jnp.int32)
    out = _embed(idx_flat, weight)
    return out.reshape(BATCH, HIST, EMBED_DIM)


# speedup vs baseline: 1.1101x; 1.0001x over previous
import jax
import jax.numpy as jnp
from jax import lax
from jax.experimental import pallas as pl
from jax.experimental.pallas import tpu as pltpu
from jax.experimental.pallas import tpu_sc as plsc

VOCAB = 1000000
EMBED_DIM = 32
BATCH = 16384
HIST = 50

NUM_CORES = 2
NUM_SUBCORES = 16
NUM_WORKERS = NUM_CORES * NUM_SUBCORES

TOTAL = BATCH * HIST
PER_WORKER = TOTAL // NUM_WORKERS
CHUNK = 1280
NCH = PER_WORKER // CHUNK

assert TOTAL % NUM_WORKERS == 0
assert PER_WORKER % CHUNK == 0, (PER_WORKER, CHUNK)


def _body(idx_hbm, table_hbm, out_hbm, idx_all, rb0, rb1,
          gsem0, gsem1, osem0, osem1):
    wid = lax.axis_index("s") * NUM_CORES + lax.axis_index("c")
    base = wid * PER_WORKER

    pltpu.sync_copy(idx_hbm.at[pl.ds(base, PER_WORKER)], idx_all)

    rb = (rb0, rb1)
    gsem = (gsem0, gsem1)
    osem = (osem0, osem1)

    def gstart(g):
        return pltpu.async_copy(
            table_hbm.at[idx_all.at[pl.ds(g * CHUNK, CHUNK)]],
            rb[g % 2], gsem[g % 2])

    def ostart(g):
        return pltpu.async_copy(
            rb[g % 2], out_hbm.at[pl.ds(base + g * CHUNK, CHUNK)],
            osem[g % 2])

    gd = [None] * NCH
    od = [None] * NCH
    gd[0] = gstart(0)
    for g in range(NCH):
        gd[g].wait()
        if g + 1 < NCH:
            if g >= 1:
                od[g - 1].wait()
            gd[g + 1] = gstart(g + 1)
        od[g] = ostart(g)
    if NCH >= 2:
        od[NCH - 2].wait()
    od[NCH - 1].wait()


@jax.jit
def _embed(idx_flat, weight):
    mesh = plsc.VectorSubcoreMesh(
        core_axis_name="c",
        subcore_axis_name="s",
        num_cores=NUM_CORES,
        num_subcores=NUM_SUBCORES,
    )
    fn = pl.kernel(
        _body,
        out_type=jax.ShapeDtypeStruct((TOTAL, EMBED_DIM), jnp.float32),
        mesh=mesh,
        scratch_types=[
            pltpu.VMEM((PER_WORKER,), jnp.int32),
            pltpu.VMEM((CHUNK, EMBED_DIM), jnp.float32),
            pltpu.VMEM((CHUNK, EMBED_DIM), jnp.float32),
            pltpu.SemaphoreType.DMA,
            pltpu.SemaphoreType.DMA,
            pltpu.SemaphoreType.DMA,
            pltpu.SemaphoreType.DMA,
        ],
        compiler_params=pltpu.CompilerParams(use_tc_tiling_on_sc=False),
    )
    return fn(idx_flat, weight)


def kernel(input_ids, weight):
    idx_flat = input_ids.reshape(-1).astype(jnp.int32)
    out = _embed(idx_flat, weight)
    return out.reshape(BATCH, HIST, EMBED_DIM)


# speedup vs baseline: 1.1112x; 1.0010x over previous
import jax
import jax.numpy as jnp
from jax import lax
from jax.experimental import pallas as pl
from jax.experimental.pallas import tpu as pltpu
from jax.experimental.pallas import tpu_sc as plsc

VOCAB = 1000000
EMBED_DIM = 32
BATCH = 16384
HIST = 50

NUM_CORES = 2
NUM_SUBCORES = 16
NUM_WORKERS = NUM_CORES * NUM_SUBCORES

TOTAL = BATCH * HIST
PER_WORKER = TOTAL // NUM_WORKERS
K = 8
CHUNK = 320
NCH = PER_WORKER // CHUNK
NSTEPS = NCH // K

assert TOTAL % NUM_WORKERS == 0
assert PER_WORKER % (CHUNK * K) == 0, (PER_WORKER, CHUNK, K)


def _body(idx_hbm, table_hbm, out_hbm, idx_all,
          rb0, rb1, rb2, rb3, rb4, rb5, rb6, rb7, gsem, osem):
    wid = lax.axis_index("s") * NUM_CORES + lax.axis_index("c")
    base = wid * PER_WORKER

    pltpu.sync_copy(idx_hbm.at[pl.ds(base, PER_WORKER)], idx_all)

    rbs = (rb0, rb1, rb2, rb3, rb4, rb5, rb6, rb7)

    @pl.loop(0, NSTEPS)
    def step(s):
        c0 = s * (K * CHUNK)
        gds = []
        for b in range(K):
            off = c0 + b * CHUNK
            gds.append(pltpu.async_copy(
                table_hbm.at[idx_all.at[pl.ds(off, CHUNK)]], rbs[b], gsem))
        for d in gds:
            d.wait()
        ods = []
        for b in range(K):
            off = c0 + b * CHUNK
            ods.append(pltpu.async_copy(
                rbs[b], out_hbm.at[pl.ds(base + off, CHUNK)], osem))
        for d in ods:
            d.wait()


@jax.jit
def _embed(idx_flat, weight):
    mesh = plsc.VectorSubcoreMesh(
        core_axis_name="c",
        subcore_axis_name="s",
        num_cores=NUM_CORES,
        num_subcores=NUM_SUBCORES,
    )
    fn = pl.kernel(
        _body,
        out_type=jax.ShapeDtypeStruct((TOTAL, EMBED_DIM), jnp.float32),
        mesh=mesh,
        scratch_types=(
            [pltpu.VMEM((PER_WORKER,), jnp.int32)]
            + [pltpu.VMEM((CHUNK, EMBED_DIM), jnp.float32) for _ in range(K)]
            + [pltpu.SemaphoreType.DMA, pltpu.SemaphoreType.DMA]
        ),
        compiler_params=pltpu.CompilerParams(use_tc_tiling_on_sc=False),
    )
    return fn(idx_flat, weight)


def kernel(input_ids, weight):
    idx_flat = input_ids.reshape(-1).astype(jnp.int32)
    out = _embed(idx_flat, weight)
    return out.reshape(BATCH, HIST, EMBED_DIM)


# speedup vs baseline: 1.4931x; 1.3437x over previous
import jax
import jax.numpy as jnp
from jax import lax
from jax.experimental import pallas as pl
from jax.experimental.pallas import tpu as pltpu
from jax.experimental.pallas import tpu_sc as plsc

VOCAB = 1000000
EMBED_DIM = 32
BATCH = 16384
HIST = 50

NUM_CORES = 2
NUM_SUBCORES = 16
NUM_WORKERS = NUM_CORES * NUM_SUBCORES

B_PER_W = BATCH // NUM_WORKERS
IDX_PER_W = B_PER_W * HIST
LANES = 16
KBLOCKS = B_PER_W // LANES


def _body(idx_hbm, table_hbm, out_hbm, idx_raw,
          gidx_a, gidx_b, rb_a, rb_b, tb_a, tb_b,
          gsem_a, gsem_b, osem_a, osem_b):
    wid = lax.axis_index("s") * NUM_CORES + lax.axis_index("c")
    b0 = wid * B_PER_W

    pltpu.sync_copy(idx_hbm.at[pl.ds(b0 * HIST, IDX_PER_W)], idx_raw)

    iota = lax.iota(jnp.int32, LANES)
    iota_h = iota * HIST

    def regroup(h, gidx):
        for kb in range(KBLOCKS):
            v = plsc.load_gather(idx_raw, [iota_h + (kb * LANES * HIST + h)])
            gidx[pl.ds(kb * LANES, LANES)] = v

    def gstart(gidx, rb, sem):
        return pltpu.async_copy(table_hbm.at[gidx], rb, sem)

    def gwait(gidx, rb, sem):
        pltpu.make_async_copy(table_hbm.at[gidx], rb, sem).wait()

    def transpose(rb, tb):
        @pl.loop(0, KBLOCKS)
        def kb_loop(kb):
            rows = iota + kb * LANES
            for d in range(EMBED_DIM):
                dv = jnp.full((LANES,), d, jnp.int32)
                v = plsc.load_gather(rb, [rows, dv])
                tb[d, pl.ds(kb * LANES, LANES)] = v

    def ostart(h, tb, sem):
        return pltpu.async_copy(
            tb, out_hbm.at[h, :, pl.ds(b0, B_PER_W)], sem)

    def owait(tb, sem):
        pltpu.make_async_copy(
            tb, out_hbm.at[0, :, pl.ds(b0, B_PER_W)], sem).wait()

    regroup(0, gidx_a)
    gstart(gidx_a, rb_a, gsem_a)

    @pl.loop(0, HIST, step=2)
    def h_loop(s):
        regroup(s + 1, gidx_b)
        gwait(gidx_a, rb_a, gsem_a)
        gstart(gidx_b, rb_b, gsem_b)

        @pl.when(s >= 2)
        def _():
            owait(tb_a, osem_a)
        transpose(rb_a, tb_a)
        ostart(s, tb_a, osem_a)

        @pl.when(s + 2 < HIST)
        def _():
            regroup(s + 2, gidx_a)
        gwait(gidx_b, rb_b, gsem_b)

        @pl.when(s + 2 < HIST)
        def _():
            gstart(gidx_a, rb_a, gsem_a)

        @pl.when(s >= 2)
        def _():
            owait(tb_b, osem_b)
        transpose(rb_b, tb_b)
        ostart(s + 1, tb_b, osem_b)

    owait(tb_a, osem_a)
    owait(tb_b, osem_b)


def _embed(idx_flat, weight):
    mesh = plsc.VectorSubcoreMesh(
        core_axis_name="c",
        subcore_axis_name="s",
        num_cores=NUM_CORES,
        num_subcores=NUM_SUBCORES,
    )
    fn = pl.kernel(
        _body,
        out_type=jax.ShapeDtypeStruct((HIST, EMBED_DIM, BATCH), jnp.float32),
        mesh=mesh,
        scratch_types=[
            pltpu.VMEM((IDX_PER_W,), jnp.int32),
            pltpu.VMEM((B_PER_W,), jnp.int32),
            pltpu.VMEM((B_PER_W,), jnp.int32),
            pltpu.VMEM((B_PER_W, EMBED_DIM), jnp.float32),
            pltpu.VMEM((B_PER_W, EMBED_DIM), jnp.float32),
            pltpu.VMEM((EMBED_DIM, B_PER_W), jnp.float32),
            pltpu.VMEM((EMBED_DIM, B_PER_W), jnp.float32),
            pltpu.SemaphoreType.DMA,
            pltpu.SemaphoreType.DMA,
            pltpu.SemaphoreType.DMA,
            pltpu.SemaphoreType.DMA,
        ],
        compiler_params=pltpu.CompilerParams(
            use_tc_tiling_on_sc=False, needs_layout_passes=False),
    )
    return fn(idx_flat, weight)


def kernel(input_ids, weight):
    idx_flat = input_ids.reshape(-1).astype(jnp.int32)
    out_t = _embed(idx_flat, weight)
    return jnp.transpose(out_t, (2, 0, 1))
